# Initial kernel scaffold; baseline (speedup 1.0000x reference)
import functools

import jax
import jax.numpy as jnp
from jax import lax
from jax.experimental import pallas as pl
from jax.experimental.pallas import tpu as pltpu
from jax.experimental.pallas import tpu_sc as plsc

B = 1024
K = 20
H = 64
NCOL = 2 * K + 1
R = NCOL * B

NW = 32
J1, C1 = 5, 128
J2, C2 = 6, 112
ROWS1 = J1 * C1
ROWS2 = J2 * C2
SAME_BASE = 20 * B


def _sc_gather(user_emb, item_emb, idx_ud, idx_us, idx_id, idx_is):
    mesh = plsc.VectorSubcoreMesh(core_axis_name="c", subcore_axis_name="s")

    @functools.partial(
        pl.kernel,
        mesh=mesh,
        out_type=(
            jax.ShapeDtypeStruct((R, H), jnp.float32),
            jax.ShapeDtypeStruct((R, H), jnp.float32),
        ),
        scratch_types=[
            pltpu.VMEM((J1, C1), jnp.int32),
            pltpu.VMEM((J2, C2), jnp.int32),
            pltpu.VMEM((ROWS1 + ROWS2, H), jnp.float32),
            pltpu.SemaphoreType.DMA,
            pltpu.SemaphoreType.DMA,
        ],
    )
    def k(uemb, iemb, i_ud, i_us, i_id, i_is, e_u, e_i, vd, vs, buf, sem_g, sem_c):
        wid = lax.axis_index("s") * 2 + lax.axis_index("c")
        for diff_idx, same_idx, diff_tab, same_tab, out in (
            (i_ud, i_us, iemb, uemb, e_u),
            (i_id, i_is, uemb, iemb, e_i),
        ):
            pltpu.sync_copy(diff_idx.at[wid], vd)
            pltpu.sync_copy(same_idx.at[wid], vs)
            handles = []
            for j in range(J1):
                handles.append(pltpu.async_copy(
                    diff_tab.at[vd.at[j]], buf.at[pl.ds(j * C1, C1)], sem_g))
            for j in range(J2):
                handles.append(pltpu.async_copy(
                    same_tab.at[vs.at[j]], buf.at[pl.ds(ROWS1 + j * C2, C2)], sem_g))
            for h in handles:
                h.wait()
            c1 = pltpu.async_copy(
                buf.at[pl.ds(0, ROWS1)],
                out.at[pl.ds(wid * ROWS1, ROWS1)], sem_c)
            c2 = pltpu.async_copy(
                buf.at[pl.ds(ROWS1, ROWS2)],
                out.at[pl.ds(SAME_BASE + wid * ROWS2, ROWS2)], sem_c)
            c1.wait()
            c2.wait()

    return k(user_emb, item_emb, idx_ud, idx_us, idx_id, idx_is)


def _tc_body(xu_ref, xi_ref, eu_ref, ei_ref, w1_ref, b1_ref, w2_ref, b2_ref,
             gw1_ref, gb1_ref, gw2_ref, gb2_ref, out_ref, acc_u, acc_i):
    t = pl.program_id(0)
    w1t = w1_ref[0:H, :]
    w1b = w1_ref[H:2 * H, :]
    b1 = b1_ref[0, :]
    w2 = w2_ref[...]
    b2 = b2_ref[0, :]

    def mlp(x, e):
        g = jnp.maximum(
            jnp.dot(x, w1t, preferred_element_type=jnp.float32)
            + jnp.dot(e, w1b, preferred_element_type=jnp.float32) + b1, 0.0)
        return jnp.maximum(
            jnp.dot(g, w2, preferred_element_type=jnp.float32) + b2, 0.0)

    h_u = mlp(xu_ref[:, 0, 0, :], eu_ref[...])
    h_i = mlp(xi_ref[:, 0, 0, :], ei_ref[...])

    @pl.when(t == 0)
    def _():
        acc_u[...] = jnp.zeros_like(acc_u)
        acc_i[...] = jnp.zeros_like(acc_i)

    @pl.when(t < 2 * K)
    def _():
        acc_u[...] += h_u
        acc_i[...] += h_i

    @pl.when(t == 2 * K)
    def _():
        gw1t = gw1_ref[0:H, :]
        gw1b = gw1_ref[H:2 * H, :]
        gb1 = gb1_ref[0, :]
        gw2 = gw2_ref[...]
        gb2 = gb2_ref[0, :]

        def trans(this, pref):
            g = jnp.maximum(
                jnp.dot(this, gw1t, preferred_element_type=jnp.float32)
                + jnp.dot(pref, gw1b, preferred_element_type=jnp.float32)
                + gb1, 0.0)
            return jnp.maximum(
                jnp.dot(g, gw2, preferred_element_type=jnp.float32) + gb2, 0.0)

        tu = trans(h_u, acc_u[...])
        ti = trans(h_i, acc_i[...])
        out_ref[...] = jnp.sum(tu * ti, axis=1)


def _tc_dense(e_u, e_i, tu4, ti4, w1, b1, w2, b2, gw1, gb1, gw2, gb2,
              interpret=False):
    small = lambda shape: pl.BlockSpec(shape, lambda t: (0,) * len(shape))
    return pl.pallas_call(
        _tc_body,
        grid=(NCOL,),
        in_specs=[
            pl.BlockSpec((B, 1, 1, H), lambda t: (0, t, 0, 0)),
            pl.BlockSpec((B, 1, 1, H), lambda t: (0, t, 0, 0)),
            pl.BlockSpec((B, H), lambda t: (t, 0)),
            pl.BlockSpec((B, H), lambda t: (t, 0)),
            small((2 * H, H)),
            small((1, H)),
            small((H, H)),
            small((1, H)),
            small((2 * H, H)),
            small((1, H)),
            small((H, H)),
            small((1, H)),
        ],
        out_specs=pl.BlockSpec((B,), lambda t: (0,)),
        out_shape=jax.ShapeDtypeStruct((B,), jnp.float32),
        scratch_shapes=[
            pltpu.VMEM((B, H), jnp.float32),
            pltpu.VMEM((B, H), jnp.float32),
        ],
        interpret=interpret,
    )(tu4, ti4, e_u, e_i, w1, b1, w2, b2, gw1, gb1, gw2, gb2)


def kernel(user_inds, item_inds, user_ne_items, user_ne_users, item_ne_users,
           item_ne_items, user_text_feats, item_text_feats, user_emb, item_emb,
           node_W1, node_b1, node_W2, node_b2, att_W1, att_b1, att_W2, att_b2,
           g_W1, g_b1, g_W2, g_b2):
    i32 = jnp.int32
    u_diff = user_ne_items.astype(i32).T.reshape(NW, J1, C1)
    u_same = jnp.concatenate(
        [user_ne_users.astype(i32).T.reshape(-1), user_inds.astype(i32)]
    ).reshape(NW, J2, C2)
    i_diff = item_ne_users.astype(i32).T.reshape(NW, J1, C1)
    i_same = jnp.concatenate(
        [item_ne_items.astype(i32).T.reshape(-1), item_inds.astype(i32)]
    ).reshape(NW, J2, C2)

    e_u, e_i = _sc_gather(user_emb, item_emb, u_diff, u_same, i_diff, i_same)

    ncols = user_text_feats.shape[1]
    tu4 = user_text_feats.reshape(B, ncols, 1, H)
    ti4 = item_text_feats.reshape(B, ncols, 1, H)
    return _tc_dense(
        e_u, e_i, tu4, ti4,
        node_W1, node_b1.reshape(1, H), node_W2, node_b2.reshape(1, H),
        g_W1, g_b1.reshape(1, H), g_W2, g_b2.reshape(1, H))


# speedup vs baseline: 1.9801x; 1.9801x over previous
import functools

import jax
import jax.numpy as jnp
from jax import lax
from jax.experimental import pallas as pl
from jax.experimental.pallas import tpu as pltpu
from jax.experimental.pallas import tpu_sc as plsc

B = 1024
K = 20
H = 64
NCOL = 2 * K + 1
S = 48

NW = 32
NPW = B // NW
NB = 256


def _sc_gather(user_emb, item_emb, idx_ud, idx_us, idx_id, idx_is):
    mesh = plsc.VectorSubcoreMesh(core_axis_name="c", subcore_axis_name="s")
    rows_w = NPW * S
    live_w = NPW * NCOL

    @functools.partial(
        pl.kernel,
        mesh=mesh,
        compiler_params=pltpu.CompilerParams(use_tc_tiling_on_sc=False),
        out_type=(
            jax.ShapeDtypeStruct((B * S, H), jnp.float32),
            jax.ShapeDtypeStruct((B * S, H), jnp.float32),
        ),
        scratch_types=[
            pltpu.VMEM((NPW, K), jnp.int32),
            pltpu.VMEM((NPW, K + 1), jnp.int32),
            pltpu.VMEM((rows_w, H), jnp.float32),
            pltpu.SemaphoreType.DMA,
        ],
    )
    def k(uemb, iemb, i_ud, i_us, i_id, i_is, e_u, e_i, vd, vs, buf, sem_g):
        wid = lax.axis_index("s") * 2 + lax.axis_index("c")
        for diff_idx, same_idx, diff_tab, same_tab, out in (
            (i_ud, i_us, iemb, uemb, e_u),
            (i_id, i_is, uemb, iemb, e_i),
        ):
            pltpu.sync_copy(diff_idx.at[wid], vd)
            pltpu.sync_copy(same_idx.at[wid], vs)

            def fire(i, carry):
                pltpu.async_copy(
                    diff_tab.at[vd.at[i]], buf.at[pl.ds(i * S, K)], sem_g)
                pltpu.async_copy(
                    same_tab.at[vs.at[i]], buf.at[pl.ds(i * S + K, K + 1)],
                    sem_g)
                return carry

            lax.fori_loop(0, NPW, fire, 0)
            pltpu.make_async_copy(
                out.at[pl.ds(0, live_w)], buf.at[pl.ds(0, live_w)],
                sem_g).wait()
            pltpu.sync_copy(buf, out.at[pl.ds(wid * rows_w, rows_w)])

    return k(user_emb, item_emb, idx_ud, idx_us, idx_id, idx_is)


def _tc_body(xu_ref, xi_ref, eu_ref, ei_ref, w1_ref, b1_ref, w2_ref, b2_ref,
             gw1_ref, gb1_ref, gw2_ref, gb2_ref, out_ref):
    w1t = w1_ref[0:H, :]
    w1b = w1_ref[H:2 * H, :]
    b1 = b1_ref[0, :]
    w2 = w2_ref[...]
    b2 = b2_ref[0, :]

    def mlp(x, e):
        g = jnp.maximum(
            jnp.dot(x, w1t, preferred_element_type=jnp.float32)
            + jnp.dot(e, w1b, preferred_element_type=jnp.float32) + b1, 0.0)
        return jnp.maximum(
            jnp.dot(g, w2, preferred_element_type=jnp.float32) + b2, 0.0)

    def side(x_ref, e_ref):
        x = x_ref[...].reshape(NB * S, H)
        h3 = mlp(x, e_ref[...]).reshape(NB, S, H)
        r8 = (h3[:, 0:8, :] + h3[:, 8:16, :] + h3[:, 16:24, :]
              + h3[:, 24:32, :] + h3[:, 32:40, :])
        pref = jnp.sum(r8, axis=1)
        this = h3[:, 40, :]
        return this, pref

    this_u, pref_u = side(xu_ref, eu_ref)
    this_i, pref_i = side(xi_ref, ei_ref)

    gw1t = gw1_ref[0:H, :]
    gw1b = gw1_ref[H:2 * H, :]
    gb1 = gb1_ref[0, :]
    gw2 = gw2_ref[...]
    gb2 = gb2_ref[0, :]

    def trans(this, pref):
        g = jnp.maximum(
            jnp.dot(this, gw1t, preferred_element_type=jnp.float32)
            + jnp.dot(pref, gw1b, preferred_element_type=jnp.float32)
            + gb1, 0.0)
        return jnp.maximum(
            jnp.dot(g, gw2, preferred_element_type=jnp.float32) + gb2, 0.0)

    tu = trans(this_u, pref_u)
    ti = trans(this_i, pref_i)
    out_ref[...] = jnp.sum(tu * ti, axis=1)


def _tc_dense(e_u, e_i, tu, ti, w1, b1, w2, b2, gw1, gb1, gw2, gb2,
              interpret=False):
    small = lambda shape: pl.BlockSpec(shape, lambda t: (0,) * len(shape))
    return pl.pallas_call(
        _tc_body,
        grid=(B // NB,),
        in_specs=[
            pl.BlockSpec((NB, S, H), lambda t: (t, 0, 0)),
            pl.BlockSpec((NB, S, H), lambda t: (t, 0, 0)),
            pl.BlockSpec((NB * S, H), lambda t: (t, 0)),
            pl.BlockSpec((NB * S, H), lambda t: (t, 0)),
            small((2 * H, H)),
            small((1, H)),
            small((H, H)),
            small((1, H)),
            small((2 * H, H)),
            small((1, H)),
            small((H, H)),
            small((1, H)),
        ],
        out_specs=pl.BlockSpec((NB,), lambda t: (t,)),
        out_shape=jax.ShapeDtypeStruct((B,), jnp.float32),
        interpret=interpret,
    )(tu, ti, e_u, e_i, w1, b1, w2, b2, gw1, gb1, gw2, gb2)


def kernel(user_inds, item_inds, user_ne_items, user_ne_users, item_ne_users,
           item_ne_items, user_text_feats, item_text_feats, user_emb, item_emb,
           node_W1, node_b1, node_W2, node_b2, att_W1, att_b1, att_W2, att_b2,
           g_W1, g_b1, g_W2, g_b2):
    i32 = jnp.int32
    u_diff = user_ne_items.astype(i32).reshape(NW, NPW, K)
    u_same = jnp.concatenate(
        [user_ne_users.astype(i32), user_inds.astype(i32)[:, None]], axis=1
    ).reshape(NW, NPW, K + 1)
    i_diff = item_ne_users.astype(i32).reshape(NW, NPW, K)
    i_same = jnp.concatenate(
        [item_ne_items.astype(i32), item_inds.astype(i32)[:, None]], axis=1
    ).reshape(NW, NPW, K + 1)

    e_u, e_i = _sc_gather(user_emb, item_emb, u_diff, u_same, i_diff, i_same)

    return _tc_dense(
        e_u, e_i, user_text_feats, item_text_feats,
        node_W1, node_b1.reshape(1, H), node_W2, node_b2.reshape(1, H),
        g_W1, g_b1.reshape(1, H), g_W2, g_b2.reshape(1, H))


# speedup vs baseline: 3.0003x; 1.5152x over previous
import functools

import jax
import jax.numpy as jnp
from jax import lax
from jax.experimental import pallas as pl
from jax.experimental.pallas import tpu as pltpu
from jax.experimental.pallas import tpu_sc as plsc

B = 1024
K = 20
H = 64
EW = 2 * H

RD = K * B
RS = (K + 1) * B
RSP = (K + 2) * B

NW = 32
J1, C1 = 5, 128
J2, C2 = 6, 112
ROWS1 = J1 * C1
ROWS2 = J2 * C2


def _sc_gather_diff(user_emb, item_emb, idx_ud, idx_id):
    mesh = plsc.VectorSubcoreMesh(core_axis_name="c", subcore_axis_name="s")

    @functools.partial(
        pl.kernel,
        mesh=mesh,
        compiler_params=pltpu.CompilerParams(use_tc_tiling_on_sc=False),
        out_type=(
            jax.ShapeDtypeStruct((RD, EW), jnp.float32),
            jax.ShapeDtypeStruct((RD, EW), jnp.float32),
        ),
        scratch_types=[
            pltpu.VMEM((J1, C1), jnp.int32),
            pltpu.VMEM((J1, C1), jnp.int32),
            pltpu.VMEM((2 * ROWS1, H), jnp.float32),
            pltpu.SemaphoreType.DMA,
        ],
    )
    def k(uemb, iemb, i_ud, i_id, e_du, e_di, vu, vi, buf, sem_g):
        wid = lax.axis_index("s") * 2 + lax.axis_index("c")
        pltpu.sync_copy(i_ud.at[wid], vu)
        pltpu.sync_copy(i_id.at[wid], vi)
        handles = []
        for j in range(J1):
            handles.append(pltpu.async_copy(
                iemb.at[vu.at[j]], buf.at[pl.ds(j * C1, C1)], sem_g))
        for j in range(J1):
            handles.append(pltpu.async_copy(
                uemb.at[vi.at[j]], buf.at[pl.ds(ROWS1 + j * C1, C1)], sem_g))
        for h in handles:
            h.wait()
        pltpu.sync_copy(
            buf.at[pl.ds(0, ROWS1)],
            e_du.at[pl.ds(wid * ROWS1, ROWS1), pl.ds(0, H)])
        pltpu.sync_copy(
            buf.at[pl.ds(ROWS1, ROWS1)],
            e_di.at[pl.ds(wid * ROWS1, ROWS1), pl.ds(0, H)])

    return k(user_emb, item_emb, idx_ud, idx_id)


def _sc_gather_same(user_emb, item_emb, idx_us, idx_is):
    mesh = plsc.VectorSubcoreMesh(core_axis_name="c", subcore_axis_name="s")

    @functools.partial(
        pl.kernel,
        mesh=mesh,
        compiler_params=pltpu.CompilerParams(use_tc_tiling_on_sc=False),
        out_type=(
            jax.ShapeDtypeStruct((RSP, EW), jnp.float32),
            jax.ShapeDtypeStruct((RSP, EW), jnp.float32),
        ),
        scratch_types=[
            pltpu.VMEM((J2, C2), jnp.int32),
            pltpu.VMEM((J2, C2), jnp.int32),
            pltpu.VMEM((2 * ROWS2, H), jnp.float32),
            pltpu.SemaphoreType.DMA,
        ],
    )
    def k(uemb, iemb, i_us, i_is, e_su, e_si, vu, vi, buf, sem_g):
        wid = lax.axis_index("s") * 2 + lax.axis_index("c")
        pltpu.sync_copy(i_us.at[wid], vu)
        pltpu.sync_copy(i_is.at[wid], vi)
        handles = []
        for j in range(J2):
            handles.append(pltpu.async_copy(
                uemb.at[vu.at[j]], buf.at[pl.ds(j * C2, C2)], sem_g))
        for j in range(J2):
            handles.append(pltpu.async_copy(
                iemb.at[vi.at[j]], buf.at[pl.ds(ROWS2 + j * C2, C2)], sem_g))
        for h in handles:
            h.wait()
        pltpu.sync_copy(
            buf.at[pl.ds(0, ROWS2)],
            e_su.at[pl.ds(wid * ROWS2, ROWS2), pl.ds(0, H)])
        pltpu.sync_copy(
            buf.at[pl.ds(ROWS2, ROWS2)],
            e_si.at[pl.ds(wid * ROWS2, ROWS2), pl.ds(0, H)])

    return k(user_emb, item_emb, idx_us, idx_is)


def _mlp_maker(w1_ref, b1_ref, w2_ref, b2_ref):
    f32 = jnp.float32
    dn_x = (((1,), (0,)), ((), ()))
    dn_e = (((1,), (1,)), ((), ()))
    dn_g = (((0,), (0,)), ((), ()))
    w1t = w1_ref[:, 0:H]
    w1b = w1_ref[:, H:2 * H]
    b1 = b1_ref[...]
    w2 = w2_ref[...]
    b2 = b2_ref[...]

    def mlp(x, e):
        g = jnp.maximum(
            lax.dot_general(w1t, x, dn_x, preferred_element_type=f32)
            + lax.dot_general(w1b, e, dn_e, preferred_element_type=f32)
            + b1, 0.0)
        return jnp.maximum(
            lax.dot_general(w2, g, dn_g, preferred_element_type=f32) + b2,
            0.0)

    return mlp, dn_x, dn_g


def _tc_diff_body(xu_ref, xi_ref, eu_ref, ei_ref, w1_ref, b1_ref, w2_ref,
                  b2_ref, accu_ref, acci_ref, acc_u, acc_i):
    t = pl.program_id(0)
    mlp, _, _ = _mlp_maker(w1_ref, b1_ref, w2_ref, b2_ref)
    h_u = mlp(xu_ref[0], eu_ref[0:B, 0:H]) + mlp(xu_ref[1], eu_ref[B:2 * B, 0:H])
    h_i = mlp(xi_ref[0], ei_ref[0:B, 0:H]) + mlp(xi_ref[1], ei_ref[B:2 * B, 0:H])

    @pl.when(t == 0)
    def _():
        acc_u[...] = jnp.zeros_like(acc_u)
        acc_i[...] = jnp.zeros_like(acc_i)

    acc_u[...] += h_u
    acc_i[...] += h_i

    @pl.when(t == K // 2 - 1)
    def _():
        accu_ref[...] = acc_u[...]
        acci_ref[...] = acc_i[...]


def _tc_final_body(xu_ref, xi_ref, eu_ref, ei_ref, pu_ref, pi_ref, w1_ref,
                   b1_ref, w2_ref, b2_ref, gw1_ref, gb1_ref, gw2_ref, gb2_ref,
                   out_ref, acc_u, acc_i):
    t = pl.program_id(0)
    f32 = jnp.float32
    mlp, dn_x, dn_g = _mlp_maker(w1_ref, b1_ref, w2_ref, b2_ref)
    h_u = mlp(xu_ref[0], eu_ref[0:B, 0:H])
    h_i = mlp(xi_ref[0], ei_ref[0:B, 0:H])
    h_u2 = mlp(xu_ref[1], eu_ref[B:2 * B, 0:H])
    h_i2 = mlp(xi_ref[1], ei_ref[B:2 * B, 0:H])

    @pl.when(t == 0)
    def _():
        acc_u[...] = pu_ref[...]
        acc_i[...] = pi_ref[...]

    @pl.when(t < K // 2)
    def _():
        acc_u[...] += h_u + h_u2
        acc_i[...] += h_i + h_i2

    @pl.when(t == K // 2)
    def _():
        gw1t = gw1_ref[:, 0:H]
        gw1b = gw1_ref[:, H:2 * H]
        gb1 = gb1_ref[...]
        gw2 = gw2_ref[...]
        gb2 = gb2_ref[...]

        def trans(this, pref):
            g = jnp.maximum(
                lax.dot_general(gw1t, this, dn_x, preferred_element_type=f32)
                + lax.dot_general(gw1b, pref, dn_x,
                                  preferred_element_type=f32)
                + gb1, 0.0)
            return jnp.maximum(
                lax.dot_general(gw2, g, dn_g, preferred_element_type=f32)
                + gb2, 0.0)

        tu = trans(h_u, acc_u[...])
        ti = trans(h_i, acc_i[...])
        out_ref[...] = jnp.sum(tu * ti, axis=0)


def _small(shape):
    return pl.BlockSpec(shape, lambda t: (0,) * len(shape))


def _tc_diff(e_du, e_di, xt_u, xt_i, w1T, b1, w2, b2, interpret=False):
    return pl.pallas_call(
        _tc_diff_body,
        grid=(K // 2,),
        in_specs=[
            pl.BlockSpec((2, H, B), lambda t: (t, 0, 0)),
            pl.BlockSpec((2, H, B), lambda t: (t, 0, 0)),
            pl.BlockSpec((2 * B, EW), lambda t: (t, 0)),
            pl.BlockSpec((2 * B, EW), lambda t: (t, 0)),
            _small((H, 2 * H)),
            _small((H, 1)),
            _small((H, H)),
            _small((H, 1)),
        ],
        out_specs=[_small((H, B)), _small((H, B))],
        out_shape=[jax.ShapeDtypeStruct((H, B), jnp.float32)] * 2,
        scratch_shapes=[
            pltpu.VMEM((H, B), jnp.float32),
            pltpu.VMEM((H, B), jnp.float32),
        ],
        interpret=interpret,
    )(xt_u, xt_i, e_du, e_di, w1T, b1, w2, b2)


def _tc_final(e_su, e_si, acc_u, acc_i, xt_u, xt_i, w1T, b1, w2, b2,
              gw1T, gb1, gw2, gb2, interpret=False):
    return pl.pallas_call(
        _tc_final_body,
        grid=(K // 2 + 1,),
        in_specs=[
            pl.BlockSpec((2, H, B), lambda t: (t + K // 2, 0, 0)),
            pl.BlockSpec((2, H, B), lambda t: (t + K // 2, 0, 0)),
            pl.BlockSpec((2 * B, EW), lambda t: (t, 0)),
            pl.BlockSpec((2 * B, EW), lambda t: (t, 0)),
            _small((H, B)),
            _small((H, B)),
            _small((H, 2 * H)),
            _small((H, 1)),
            _small((H, H)),
            _small((H, 1)),
            _small((H, 2 * H)),
            _small((H, 1)),
            _small((H, H)),
            _small((H, 1)),
        ],
        out_specs=pl.BlockSpec((B,), lambda t: (0,)),
        out_shape=jax.ShapeDtypeStruct((B,), jnp.float32),
        scratch_shapes=[
            pltpu.VMEM((H, B), jnp.float32),
            pltpu.VMEM((H, B), jnp.float32),
        ],
        interpret=interpret,
    )(xt_u, xt_i, e_su, e_si, acc_u, acc_i, w1T, b1, w2, b2,
      gw1T, gb1, gw2, gb2)


def kernel(user_inds, item_inds, user_ne_items, user_ne_users, item_ne_users,
           item_ne_items, user_text_feats, item_text_feats, user_emb, item_emb,
           node_W1, node_b1, node_W2, node_b2, att_W1, att_b1, att_W2, att_b2,
           g_W1, g_b1, g_W2, g_b2):
    i32 = jnp.int32
    u_diff = user_ne_items.astype(i32).T.reshape(NW, J1, C1)
    i_diff = item_ne_users.astype(i32).T.reshape(NW, J1, C1)
    u_same = jnp.concatenate(
        [user_ne_users.astype(i32).T.reshape(-1), user_inds.astype(i32)]
    ).reshape(NW, J2, C2)
    i_same = jnp.concatenate(
        [item_ne_items.astype(i32).T.reshape(-1), item_inds.astype(i32)]
    ).reshape(NW, J2, C2)

    e_du, e_di = _sc_gather_diff(user_emb, item_emb, u_diff, i_diff)
    e_su, e_si = _sc_gather_same(user_emb, item_emb, u_same, i_same)

    xt_u = jnp.transpose(user_text_feats, (1, 2, 0))
    xt_i = jnp.transpose(item_text_feats, (1, 2, 0))
    w1T = node_W1.T
    gw1T = g_W1.T
    b1 = node_b1.reshape(H, 1)
    b2 = node_b2.reshape(H, 1)
    gb1 = g_b1.reshape(H, 1)
    gb2 = g_b2.reshape(H, 1)

    acc_u, acc_i = _tc_diff(e_du, e_di, xt_u, xt_i, w1T, b1, node_W2, b2)
    return _tc_final(e_su, e_si, acc_u, acc_i, xt_u, xt_i, w1T, b1,
                     node_W2, b2, gw1T, gb1, g_W2, gb2)
